# initial kernel scaffold (unmeasured)
import jax
import jax.numpy as jnp
from jax import lax
from jax.experimental import pallas as pl
from jax.experimental.pallas import tpu as pltpu

N_DEV = 4
M = 4096
N = 8192
HALF = M // 2
CH = HALF // N_DEV
NT = 2048
F32 = jnp.float32
BF16 = jnp.bfloat16


def kernel(x, w_mat):
    def body(x_ref, w_ref, out_ref, comm_a, comm_b,
             send_a, recv_a, send_b, recv_b,
             credit_a, credit_b, out_sem):
        p = lax.axis_index("i")
        right = lax.rem(p + 1, N_DEV)
        left = lax.rem(p + N_DEV - 1, N_DEV)

        def mod4(v):
            return lax.rem(v + 2 * N_DEV, N_DEV)

        def compute_sum(ring, c, slot, add):
            comm = comm_a if ring == 0 else comm_b
            base = 0 if ring == 0 else HALF
            xc = x_ref[pl.ds(base + c * CH, CH), :]
            for j in range(N // NT):
                part = jnp.dot(
                    xc, w_ref[:, j * NT:(j + 1) * NT],
                    preferred_element_type=F32,
                )
                if add:
                    part = part + comm[slot, :, j * NT:(j + 1) * NT].astype(F32)
                comm[slot, :, j * NT:(j + 1) * NT] = part.astype(BF16)

        def silu_store(ring, c, slot):
            comm = comm_a if ring == 0 else comm_b
            base = 0 if ring == 0 else HALF
            for j in range(N // NT):
                y = comm[slot, :, j * NT:(j + 1) * NT].astype(F32)
                z = y / (1.0 + jnp.exp(-y))
                comm[slot, :, j * NT:(j + 1) * NT] = z.astype(BF16)
            cp = pltpu.make_async_copy(
                comm.at[slot],
                out_ref.at[pl.ds(base + c * CH, CH), :],
                out_sem,
            )
            cp.start()
            cp.wait()

        def ring_rdma(g, ring):
            comm, ssem, rsem, peer = (
                (comm_a, send_a, recv_a, right) if ring == 0
                else (comm_b, send_b, recv_b, left)
            )
            return pltpu.make_async_remote_copy(
                src_ref=comm.at[g % 2],
                dst_ref=comm.at[(g + 1) % 2],
                send_sem=ssem.at[g % 2],
                recv_sem=rsem.at[(g + 1) % 2],
                device_id=(peer,),
                device_id_type=pl.DeviceIdType.MESH,
            )

        compute_sum(0, p, 0, add=False)
        compute_sum(1, p, 0, add=False)

        bsem = pltpu.get_barrier_semaphore()
        for nbr in (left, right):
            pl.semaphore_signal(
                bsem, inc=1, device_id=(nbr,),
                device_id_type=pl.DeviceIdType.MESH,
            )
        pl.semaphore_wait(bsem, 2)

        for g in range(6):
            if g >= 1:
                pl.semaphore_wait(credit_a, 1)
                pl.semaphore_wait(credit_b, 1)
            ra = ring_rdma(g, 0)
            rb = ring_rdma(g, 1)
            ra.start()
            rb.start()
            ra.wait()
            rb.wait()
            slot = (g + 1) % 2
            if g <= 2:
                pl.semaphore_signal(
                    credit_a, inc=1, device_id=(left,),
                    device_id_type=pl.DeviceIdType.MESH,
                )
                pl.semaphore_signal(
                    credit_b, inc=1, device_id=(right,),
                    device_id_type=pl.DeviceIdType.MESH,
                )
                compute_sum(0, mod4(p - g - 1), slot, add=True)
                compute_sum(1, mod4(p + g + 1), slot, add=True)
            else:
                s = g - 3
                silu_store(0, mod4(p + 1 - s), g % 2)
                silu_store(1, mod4(p - 1 + s), g % 2)
                if g <= 4:
                    pl.semaphore_signal(
                        credit_a, inc=1, device_id=(left,),
                        device_id_type=pl.DeviceIdType.MESH,
                    )
                    pl.semaphore_signal(
                        credit_b, inc=1, device_id=(right,),
                        device_id_type=pl.DeviceIdType.MESH,
                    )
        silu_store(0, mod4(p - 2), 0)
        silu_store(1, mod4(p + 2), 0)

    return pl.pallas_call(
        body,
        out_shape=jax.ShapeDtypeStruct((M, N), BF16),
        in_specs=[
            pl.BlockSpec(memory_space=pltpu.VMEM),
            pl.BlockSpec(memory_space=pltpu.VMEM),
        ],
        out_specs=pl.BlockSpec(memory_space=pltpu.ANY),
        scratch_shapes=[
            pltpu.VMEM((2, CH, N), BF16),
            pltpu.VMEM((2, CH, N), BF16),
            pltpu.SemaphoreType.DMA((2,)),
            pltpu.SemaphoreType.DMA((2,)),
            pltpu.SemaphoreType.DMA((2,)),
            pltpu.SemaphoreType.DMA((2,)),
            pltpu.SemaphoreType.REGULAR,
            pltpu.SemaphoreType.REGULAR,
            pltpu.SemaphoreType.DMA,
        ],
        compiler_params=pltpu.CompilerParams(collective_id=0),
    )(x, w_mat)


# baseline (device time: 782187 ns/iter reference)
import jax
import jax.numpy as jnp
from jax import lax
from jax.experimental import pallas as pl
from jax.experimental.pallas import tpu as pltpu

N_DEV = 4
M = 4096
N = 8192
HALF = M // 2
CH = HALF // N_DEV
NT = 1024
F32 = jnp.float32
BF16 = jnp.bfloat16


def kernel(x, w_mat):
    def body(x_ref, w_ref, out_ref, comm_a, comm_b,
             send_a, recv_a, send_b, recv_b,
             credit_a, credit_b, out_sem):
        p = lax.axis_index("i")
        right = lax.rem(p + 1, N_DEV)
        left = lax.rem(p + N_DEV - 1, N_DEV)

        def mod4(v):
            return lax.rem(v + 2 * N_DEV, N_DEV)

        def compute_sum(ring, c, slot, add):
            comm = comm_a if ring == 0 else comm_b
            base = 0 if ring == 0 else HALF
            xc = x_ref[pl.ds(base + c * CH, CH), :]

            def tile(j, _):
                cols = pl.ds(pl.multiple_of(j * NT, NT), NT)
                part = jnp.dot(xc, w_ref[:, cols], preferred_element_type=F32)
                if add:
                    part = part + comm[slot, :, cols].astype(F32)
                comm[slot, :, cols] = part.astype(BF16)
                return 0

            lax.fori_loop(0, N // NT, tile, 0)

        def silu_store(ring, c, slot):
            comm = comm_a if ring == 0 else comm_b
            base = 0 if ring == 0 else HALF
            def tile(j, _):
                cols = pl.ds(pl.multiple_of(j * NT, NT), NT)
                y = comm[slot, :, cols].astype(F32)
                comm[slot, :, cols] = (y / (1.0 + jnp.exp(-y))).astype(BF16)
                return 0

            lax.fori_loop(0, N // NT, tile, 0)
            cp = pltpu.make_async_copy(
                comm.at[slot],
                out_ref.at[pl.ds(base + c * CH, CH), :],
                out_sem,
            )
            cp.start()
            cp.wait()

        def ring_rdma(g, ring):
            comm, ssem, rsem, peer = (
                (comm_a, send_a, recv_a, right) if ring == 0
                else (comm_b, send_b, recv_b, left)
            )
            return pltpu.make_async_remote_copy(
                src_ref=comm.at[g % 2],
                dst_ref=comm.at[(g + 1) % 2],
                send_sem=ssem.at[g % 2],
                recv_sem=rsem.at[(g + 1) % 2],
                device_id=(peer,),
                device_id_type=pl.DeviceIdType.MESH,
            )

        compute_sum(0, p, 0, add=False)
        compute_sum(1, p, 0, add=False)

        bsem = pltpu.get_barrier_semaphore()
        for nbr in (left, right):
            pl.semaphore_signal(
                bsem, inc=1, device_id=(nbr,),
                device_id_type=pl.DeviceIdType.MESH,
            )
        pl.semaphore_wait(bsem, 2)

        for g in range(6):
            if g >= 1:
                pl.semaphore_wait(credit_a, 1)
                pl.semaphore_wait(credit_b, 1)
            ra = ring_rdma(g, 0)
            rb = ring_rdma(g, 1)
            ra.start()
            rb.start()
            ra.wait()
            rb.wait()
            slot = (g + 1) % 2
            if g <= 2:
                pl.semaphore_signal(
                    credit_a, inc=1, device_id=(left,),
                    device_id_type=pl.DeviceIdType.MESH,
                )
                pl.semaphore_signal(
                    credit_b, inc=1, device_id=(right,),
                    device_id_type=pl.DeviceIdType.MESH,
                )
                compute_sum(0, mod4(p - g - 1), slot, add=True)
                compute_sum(1, mod4(p + g + 1), slot, add=True)
            else:
                s = g - 3
                silu_store(0, mod4(p + 1 - s), g % 2)
                silu_store(1, mod4(p - 1 + s), g % 2)
                if g <= 4:
                    pl.semaphore_signal(
                        credit_a, inc=1, device_id=(left,),
                        device_id_type=pl.DeviceIdType.MESH,
                    )
                    pl.semaphore_signal(
                        credit_b, inc=1, device_id=(right,),
                        device_id_type=pl.DeviceIdType.MESH,
                    )
        silu_store(0, mod4(p - 2), 0)
        silu_store(1, mod4(p + 2), 0)

    return pl.pallas_call(
        body,
        out_shape=jax.ShapeDtypeStruct((M, N), BF16),
        in_specs=[
            pl.BlockSpec(memory_space=pltpu.VMEM),
            pl.BlockSpec(memory_space=pltpu.VMEM),
        ],
        out_specs=pl.BlockSpec(memory_space=pl.ANY),
        scratch_shapes=[
            pltpu.VMEM((2, CH, N), BF16),
            pltpu.VMEM((2, CH, N), BF16),
            pltpu.SemaphoreType.DMA((2,)),
            pltpu.SemaphoreType.DMA((2,)),
            pltpu.SemaphoreType.DMA((2,)),
            pltpu.SemaphoreType.DMA((2,)),
            pltpu.SemaphoreType.REGULAR,
            pltpu.SemaphoreType.REGULAR,
            pltpu.SemaphoreType.DMA,
        ],
        compiler_params=pltpu.CompilerParams(
            collective_id=0,
            vmem_limit_bytes=100 * 1024 * 1024,
        ),
    )(x.astype(BF16), w_mat.astype(BF16))


# device time: 664713 ns/iter; 1.1767x vs baseline; 1.1767x over previous
import jax
import jax.numpy as jnp
from jax import lax
from jax.experimental import pallas as pl
from jax.experimental.pallas import tpu as pltpu

N_DEV = 4
M = 4096
N = 8192
HALF = M // 2
CH = HALF // N_DEV
HR = CH // 2
NT = 1024
NK = 12
F32 = jnp.float32
BF16 = jnp.bfloat16


def kernel(x, w_mat):
    def body(x_ref, w_ref, out_ref, comm_a, comm_b,
             send_a, recv_a, send_b, recv_b,
             credit_a, credit_b, out_sem):
        p = lax.axis_index("i")
        right = lax.rem(p + 1, N_DEV)
        left = lax.rem(p + N_DEV - 1, N_DEV)

        def mod4(v):
            return lax.rem(v + 2 * N_DEV, N_DEV)

        def ring(r):
            if r == 0:
                return comm_a, send_a, recv_a, credit_a, right, left
            return comm_b, send_b, recv_b, credit_b, left, right

        def base(r):
            return 0 if r == 0 else HALF

        def rdma(r, k):
            comm, ssem, rsem, _, peer, _ = ring(r)
            g, h = k // 2, k % 2
            rows = pl.ds(h * HR, HR)
            return pltpu.make_async_remote_copy(
                src_ref=comm.at[g % 2, rows, :],
                dst_ref=comm.at[(g + 1) % 2, rows, :],
                send_sem=ssem.at[g % 2, h],
                recv_sem=rsem.at[(g + 1) % 2, h],
                device_id=(peer,),
                device_id_type=pl.DeviceIdType.MESH,
            )

        def compute_sum(r, c, slot, h, add):
            comm = ring(r)[0]
            xc = x_ref[pl.ds(base(r) + c * CH + h * HR, HR), :]

            def tile(j, _):
                cols = pl.ds(pl.multiple_of(j * NT, NT), NT)
                part = jnp.dot(xc, w_ref[:, cols], preferred_element_type=F32)
                if add:
                    part = part + comm[slot, h * HR:(h + 1) * HR, cols].astype(F32)
                comm[slot, h * HR:(h + 1) * HR, cols] = part.astype(BF16)
                return 0

            lax.fori_loop(0, N // NT, tile, 0)

        def silu_store(r, c, slot, h):
            comm = ring(r)[0]

            def tile(j, _):
                cols = pl.ds(pl.multiple_of(j * NT, NT), NT)
                y = comm[slot, h * HR:(h + 1) * HR, cols].astype(F32)
                comm[slot, h * HR:(h + 1) * HR, cols] = (
                    y / (1.0 + jnp.exp(-y))
                ).astype(BF16)
                return 0

            lax.fori_loop(0, N // NT, tile, 0)
            cp = pltpu.make_async_copy(
                comm.at[slot, pl.ds(h * HR, HR), :],
                out_ref.at[pl.ds(base(r) + c * CH + h * HR, HR), :],
                out_sem,
            )
            cp.start()
            cp.wait()

        for h in (0, 1):
            compute_sum(0, p, 0, h, add=False)
            compute_sum(1, p, 0, h, add=False)

        bsem = pltpu.get_barrier_semaphore()
        for nbr in (left, right):
            pl.semaphore_signal(
                bsem, inc=1, device_id=(nbr,),
                device_id_type=pl.DeviceIdType.MESH,
            )
        pl.semaphore_wait(bsem, 2)

        for k in (0, 1):
            rdma(0, k).start()
            rdma(1, k).start()

        for k in range(NK):
            g, h = k // 2, k % 2
            for r in (0, 1):
                comm, _, _, credit, _, peer_in = ring(r)
                d = rdma(r, k)
                d.wait_recv()
                if g <= 2:
                    c_in = mod4(p - g - 1) if r == 0 else mod4(p + g + 1)
                    compute_sum(r, c_in, (g + 1) % 2, h, add=True)
                elif g == 5:
                    c_fin = mod4(p - 2) if r == 0 else mod4(p + 2)
                    silu_store(r, c_fin, (g + 1) % 2, h)
                d.wait_send()
                if g >= 3:
                    c_sent = mod4(p + 4 - g) if r == 0 else mod4(p + g - 4)
                    silu_store(r, c_sent, g % 2, h)
                if k < NK - 2:
                    pl.semaphore_signal(
                        credit, inc=1, device_id=(peer_in,),
                        device_id_type=pl.DeviceIdType.MESH,
                    )
                if k + 2 < NK:
                    pl.semaphore_wait(credit, 1)
                    rdma(r, k + 2).start()

    return pl.pallas_call(
        body,
        out_shape=jax.ShapeDtypeStruct((M, N), BF16),
        in_specs=[
            pl.BlockSpec(memory_space=pltpu.VMEM),
            pl.BlockSpec(memory_space=pltpu.VMEM),
        ],
        out_specs=pl.BlockSpec(memory_space=pl.ANY),
        scratch_shapes=[
            pltpu.VMEM((2, CH, N), BF16),
            pltpu.VMEM((2, CH, N), BF16),
            pltpu.SemaphoreType.DMA((2, 2)),
            pltpu.SemaphoreType.DMA((2, 2)),
            pltpu.SemaphoreType.DMA((2, 2)),
            pltpu.SemaphoreType.DMA((2, 2)),
            pltpu.SemaphoreType.REGULAR,
            pltpu.SemaphoreType.REGULAR,
            pltpu.SemaphoreType.DMA,
        ],
        compiler_params=pltpu.CompilerParams(
            collective_id=0,
            vmem_limit_bytes=100 * 1024 * 1024,
        ),
    )(x.astype(BF16), w_mat.astype(BF16))


# device time: 645447 ns/iter; 1.2119x vs baseline; 1.0298x over previous
import jax
import jax.numpy as jnp
from jax import lax
from jax.experimental import pallas as pl
from jax.experimental.pallas import tpu as pltpu

N_DEV = 4
M = 4096
N = 8192
HALF = M // 2
CH = HALF // N_DEV
HR = CH // 2
NT = 1024
NK = 12
F32 = jnp.float32
BF16 = jnp.bfloat16


def kernel(x, w_mat):
    def body(x_ref, w_ref, out_ref, comm_a, comm_b,
             send_a, recv_a, send_b, recv_b,
             credit_a, credit_b, out_sem):
        p = lax.axis_index("i")
        right = lax.rem(p + 1, N_DEV)
        left = lax.rem(p + N_DEV - 1, N_DEV)

        def mod4(v):
            return lax.rem(v + 2 * N_DEV, N_DEV)

        def ring(r):
            if r == 0:
                return comm_a, send_a, recv_a, credit_a, right, left
            return comm_b, send_b, recv_b, credit_b, left, right

        def base(r):
            return 0 if r == 0 else HALF

        def rdma(r, k):
            comm, ssem, rsem, _, peer, _ = ring(r)
            g, h = k // 2, k % 2
            rows = pl.ds(h * HR, HR)
            return pltpu.make_async_remote_copy(
                src_ref=comm.at[g % 2, rows, :],
                dst_ref=comm.at[(g + 1) % 2, rows, :],
                send_sem=ssem.at[g % 2, h],
                recv_sem=rsem.at[(g + 1) % 2, h],
                device_id=(peer,),
                device_id_type=pl.DeviceIdType.MESH,
            )

        def compute_sum(r, c, slot, h, add):
            comm = ring(r)[0]
            xc = x_ref[pl.ds(base(r) + c * CH + h * HR, HR), :]

            def tile(j, _):
                cols = pl.ds(pl.multiple_of(j * NT, NT), NT)
                part = jnp.dot(xc, w_ref[:, cols], preferred_element_type=F32)
                if add:
                    part = part + comm[slot, h * HR:(h + 1) * HR, cols].astype(F32)
                comm[slot, h * HR:(h + 1) * HR, cols] = part.astype(BF16)
                return 0

            lax.fori_loop(0, N // NT, tile, 0)

        def silu_store(r, c, slot, h):
            comm = ring(r)[0]

            def tile(j, _):
                cols = pl.ds(pl.multiple_of(j * NT, NT), NT)
                y = comm[slot, h * HR:(h + 1) * HR, cols].astype(F32)
                comm[slot, h * HR:(h + 1) * HR, cols] = (
                    y / (1.0 + jnp.exp(-y))
                ).astype(BF16)
                return 0

            lax.fori_loop(0, N // NT, tile, 0)
            cp = pltpu.make_async_copy(
                comm.at[slot, pl.ds(h * HR, HR), :],
                out_ref.at[pl.ds(base(r) + c * CH + h * HR, HR), :],
                out_sem,
            )
            cp.start()
            cp.wait()

        bsem = pltpu.get_barrier_semaphore()
        for nbr in (left, right):
            pl.semaphore_signal(
                bsem, inc=1, device_id=(nbr,),
                device_id_type=pl.DeviceIdType.MESH,
            )
        pl.semaphore_wait(bsem, 2)

        for k in (0, 1):
            for r in (0, 1):
                compute_sum(r, p, 0, k, add=False)
                rdma(r, k).start()

        for k in range(NK):
            g, h = k // 2, k % 2
            for r in (0, 1):
                comm, _, _, credit, _, peer_in = ring(r)
                d = rdma(r, k)
                d.wait_recv()
                if g <= 2:
                    c_in = mod4(p - g - 1) if r == 0 else mod4(p + g + 1)
                    compute_sum(r, c_in, (g + 1) % 2, h, add=True)
                elif g == 5:
                    c_fin = mod4(p - 2) if r == 0 else mod4(p + 2)
                    silu_store(r, c_fin, (g + 1) % 2, h)
                d.wait_send()
                if g >= 3:
                    c_sent = mod4(p + 4 - g) if r == 0 else mod4(p + g - 4)
                    silu_store(r, c_sent, g % 2, h)
                if k < NK - 2:
                    pl.semaphore_signal(
                        credit, inc=1, device_id=(peer_in,),
                        device_id_type=pl.DeviceIdType.MESH,
                    )
                if k + 2 < NK:
                    pl.semaphore_wait(credit, 1)
                    rdma(r, k + 2).start()

    return pl.pallas_call(
        body,
        out_shape=jax.ShapeDtypeStruct((M, N), BF16),
        in_specs=[
            pl.BlockSpec(memory_space=pltpu.VMEM),
            pl.BlockSpec(memory_space=pltpu.VMEM),
        ],
        out_specs=pl.BlockSpec(memory_space=pl.ANY),
        scratch_shapes=[
            pltpu.VMEM((2, CH, N), BF16),
            pltpu.VMEM((2, CH, N), BF16),
            pltpu.SemaphoreType.DMA((2, 2)),
            pltpu.SemaphoreType.DMA((2, 2)),
            pltpu.SemaphoreType.DMA((2, 2)),
            pltpu.SemaphoreType.DMA((2, 2)),
            pltpu.SemaphoreType.REGULAR,
            pltpu.SemaphoreType.REGULAR,
            pltpu.SemaphoreType.DMA,
        ],
        compiler_params=pltpu.CompilerParams(
            collective_id=0,
            vmem_limit_bytes=100 * 1024 * 1024,
        ),
    )(x.astype(BF16), w_mat.astype(BF16))


# device time: 622580 ns/iter; 1.2564x vs baseline; 1.0367x over previous
import jax
import jax.numpy as jnp
from jax import lax
from jax.experimental import pallas as pl
from jax.experimental.pallas import tpu as pltpu

N_DEV = 4
M = 4096
K = 1024
N = 8192
HALF = M // 2
CH = HALF // N_DEV
HR = CH // 2
NT = 1024
WT = 128
NK = 12
F32 = jnp.float32
BF16 = jnp.bfloat16


def kernel(x, w_mat):
    def body(x_hbm, w_hbm, out_ref, comm_a, comm_b, wb, wstage, xstage,
             send_a, recv_a, send_b, recv_b,
             credit_a, credit_b, out_sem, cvt_sems, x_sem):
        p = lax.axis_index("i")
        right = lax.rem(p + 1, N_DEV)
        left = lax.rem(p + N_DEV - 1, N_DEV)

        def mod4(v):
            return lax.rem(v + 2 * N_DEV, N_DEV)

        def ring(r):
            if r == 0:
                return comm_a, send_a, recv_a, credit_a, right, left
            return comm_b, send_b, recv_b, credit_b, left, right

        def base(r):
            return 0 if r == 0 else HALF

        def rdma(r, k):
            comm, ssem, rsem, _, peer, _ = ring(r)
            g, h = k // 2, k % 2
            rows = pl.ds(h * HR, HR)
            return pltpu.make_async_remote_copy(
                src_ref=comm.at[g % 2, rows, :],
                dst_ref=comm.at[(g + 1) % 2, rows, :],
                send_sem=ssem.at[g % 2, h],
                recv_sem=rsem.at[(g + 1) % 2, h],
                device_id=(peer,),
                device_id_type=pl.DeviceIdType.MESH,
            )

        n_wt = K // WT
        cps = []
        for t in range(n_wt):
            cp = pltpu.make_async_copy(
                w_hbm.at[pl.ds(t * WT, WT), :],
                wstage.at[t % 2],
                cvt_sems.at[t % 2],
            )
            cp.start()
            cps.append(cp)
            if t >= 1:
                cps[t - 1].wait()
                wb[pl.ds((t - 1) * WT, WT), :] = wstage[(t - 1) % 2].astype(BF16)
        cps[n_wt - 1].wait()
        wb[pl.ds((n_wt - 1) * WT, WT), :] = wstage[(n_wt - 1) % 2].astype(BF16)

        def compute_sum(r, c, slot, h, add):
            comm = ring(r)[0]
            row0 = base(r) + c * CH + h * HR
            cp = pltpu.make_async_copy(
                x_hbm.at[pl.ds(row0, HR), :], xstage, x_sem,
            )
            cp.start()
            cp.wait()
            xc = xstage[...].astype(BF16)

            def tile(j, _):
                cols = pl.ds(pl.multiple_of(j * NT, NT), NT)
                part = jnp.dot(xc, wb[:, cols], preferred_element_type=F32)
                if add:
                    part = part + comm[slot, h * HR:(h + 1) * HR, cols].astype(F32)
                comm[slot, h * HR:(h + 1) * HR, cols] = part.astype(BF16)
                return 0

            lax.fori_loop(0, N // NT, tile, 0)

        def silu_store(r, c, slot, h):
            comm = ring(r)[0]

            def tile(j, _):
                cols = pl.ds(pl.multiple_of(j * NT, NT), NT)
                y = comm[slot, h * HR:(h + 1) * HR, cols].astype(F32)
                comm[slot, h * HR:(h + 1) * HR, cols] = (
                    y / (1.0 + jnp.exp(-y))
                ).astype(BF16)
                return 0

            lax.fori_loop(0, N // NT, tile, 0)
            cp = pltpu.make_async_copy(
                comm.at[slot, pl.ds(h * HR, HR), :],
                out_ref.at[pl.ds(base(r) + c * CH + h * HR, HR), :],
                out_sem,
            )
            cp.start()
            cp.wait()

        bsem = pltpu.get_barrier_semaphore()
        for nbr in (left, right):
            pl.semaphore_signal(
                bsem, inc=1, device_id=(nbr,),
                device_id_type=pl.DeviceIdType.MESH,
            )
        pl.semaphore_wait(bsem, 2)

        for k in (0, 1):
            for r in (0, 1):
                compute_sum(r, p, 0, k, add=False)
                rdma(r, k).start()

        for k in range(NK):
            g, h = k // 2, k % 2
            for r in (0, 1):
                comm, _, _, credit, _, peer_in = ring(r)
                d = rdma(r, k)
                d.wait_recv()
                if g <= 2:
                    c_in = mod4(p - g - 1) if r == 0 else mod4(p + g + 1)
                    compute_sum(r, c_in, (g + 1) % 2, h, add=True)
                elif g == 5:
                    c_fin = mod4(p - 2) if r == 0 else mod4(p + 2)
                    silu_store(r, c_fin, (g + 1) % 2, h)
                d.wait_send()
                if g >= 3:
                    c_sent = mod4(p + 4 - g) if r == 0 else mod4(p + g - 4)
                    silu_store(r, c_sent, g % 2, h)
                if k < NK - 2:
                    pl.semaphore_signal(
                        credit, inc=1, device_id=(peer_in,),
                        device_id_type=pl.DeviceIdType.MESH,
                    )
                if k + 2 < NK:
                    pl.semaphore_wait(credit, 1)
                    rdma(r, k + 2).start()

    return pl.pallas_call(
        body,
        out_shape=jax.ShapeDtypeStruct((M, N), BF16),
        in_specs=[
            pl.BlockSpec(memory_space=pl.ANY),
            pl.BlockSpec(memory_space=pl.ANY),
        ],
        out_specs=pl.BlockSpec(memory_space=pl.ANY),
        scratch_shapes=[
            pltpu.VMEM((2, CH, N), BF16),
            pltpu.VMEM((2, CH, N), BF16),
            pltpu.VMEM((K, N), BF16),
            pltpu.VMEM((2, WT, N), F32),
            pltpu.VMEM((HR, K), F32),
            pltpu.SemaphoreType.DMA((2, 2)),
            pltpu.SemaphoreType.DMA((2, 2)),
            pltpu.SemaphoreType.DMA((2, 2)),
            pltpu.SemaphoreType.DMA((2, 2)),
            pltpu.SemaphoreType.REGULAR,
            pltpu.SemaphoreType.REGULAR,
            pltpu.SemaphoreType.DMA,
            pltpu.SemaphoreType.DMA((2,)),
            pltpu.SemaphoreType.DMA,
        ],
        compiler_params=pltpu.CompilerParams(
            collective_id=0,
            vmem_limit_bytes=100 * 1024 * 1024,
        ),
    )(x, w_mat)


# device time: 617799 ns/iter; 1.2661x vs baseline; 1.0077x over previous
import jax
import jax.numpy as jnp
from jax import lax
from jax.experimental import pallas as pl
from jax.experimental.pallas import tpu as pltpu

N_DEV = 4
M = 4096
K = 1024
N = 8192
HALF = M // 2
CH = HALF // N_DEV
HR = CH // 2
NT = 1024
WT = 128
NK = 12
F32 = jnp.float32
BF16 = jnp.bfloat16


def kernel(x, w_mat):
    def body(x_hbm, w_hbm, out_ref, comm_a, comm_b, wb, wstage, xstage, sbuf,
             send_a, recv_a, send_b, recv_b,
             credit_a, credit_b, out_sem, cvt_sems, x_sem):
        p = lax.axis_index("i")
        right = lax.rem(p + 1, N_DEV)
        left = lax.rem(p + N_DEV - 1, N_DEV)

        def mod4(v):
            return lax.rem(v + 2 * N_DEV, N_DEV)

        def ring(r):
            if r == 0:
                return comm_a, send_a, recv_a, credit_a, right, left
            return comm_b, send_b, recv_b, credit_b, left, right

        def base(r):
            return 0 if r == 0 else HALF

        def rdma(r, k):
            comm, ssem, rsem, _, peer, _ = ring(r)
            g, h = k // 2, k % 2
            rows = pl.ds(h * HR, HR)
            return pltpu.make_async_remote_copy(
                src_ref=comm.at[g % 2, rows, :],
                dst_ref=comm.at[(g + 1) % 2, rows, :],
                send_sem=ssem.at[g % 2, h],
                recv_sem=rsem.at[(g + 1) % 2, h],
                device_id=(peer,),
                device_id_type=pl.DeviceIdType.MESH,
            )

        n_wt = K // WT
        cps = []
        for t in range(n_wt):
            cp = pltpu.make_async_copy(
                w_hbm.at[pl.ds(t * WT, WT), :],
                wstage.at[t % 2],
                cvt_sems.at[t % 2],
            )
            cp.start()
            cps.append(cp)
            if t >= 1:
                cps[t - 1].wait()
                wb[pl.ds((t - 1) * WT, WT), :] = wstage[(t - 1) % 2].astype(BF16)
        cps[n_wt - 1].wait()
        wb[pl.ds((n_wt - 1) * WT, WT), :] = wstage[(n_wt - 1) % 2].astype(BF16)

        def compute_sum(r, c, slot, h, add):
            comm = ring(r)[0]
            row0 = base(r) + c * CH + h * HR
            cp = pltpu.make_async_copy(
                x_hbm.at[pl.ds(row0, HR), :], xstage, x_sem,
            )
            cp.start()
            cp.wait()
            xc = xstage[...].astype(BF16)

            def tile(j, _):
                cols = pl.ds(pl.multiple_of(j * NT, NT), NT)
                part = jnp.dot(xc, wb[:, cols], preferred_element_type=F32)
                if add:
                    part = part + comm[slot, h * HR:(h + 1) * HR, cols].astype(F32)
                comm[slot, h * HR:(h + 1) * HR, cols] = part.astype(BF16)
                return 0

            lax.fori_loop(0, N // NT, tile, 0)

        def silu_store(r, c, slot, h):
            comm = ring(r)[0]

            def tile(j, _):
                cols = pl.ds(pl.multiple_of(j * NT, NT), NT)
                y = comm[slot, h * HR:(h + 1) * HR, cols].astype(F32)
                sbuf[:, cols] = (y / (1.0 + jnp.exp(-y))).astype(BF16)
                return 0

            lax.fori_loop(0, N // NT, tile, 0)
            cp = pltpu.make_async_copy(
                sbuf,
                out_ref.at[pl.ds(base(r) + c * CH + h * HR, HR), :],
                out_sem,
            )
            cp.start()
            cp.wait()

        bsem = pltpu.get_barrier_semaphore()
        for nbr in (left, right):
            pl.semaphore_signal(
                bsem, inc=1, device_id=(nbr,),
                device_id_type=pl.DeviceIdType.MESH,
            )
        pl.semaphore_wait(bsem, 2)

        for k in (0, 1):
            for r in (0, 1):
                compute_sum(r, p, 0, k, add=False)
                rdma(r, k).start()

        for k in range(NK):
            g, h = k // 2, k % 2
            for r in (0, 1):
                comm, _, _, credit, _, peer_in = ring(r)
                d = rdma(r, k)
                d.wait_recv()
                if g <= 2:
                    c_in = mod4(p - g - 1) if r == 0 else mod4(p + g + 1)
                    compute_sum(r, c_in, (g + 1) % 2, h, add=True)
                    if g == 2:
                        c_own = mod4(p + 1) if r == 0 else mod4(p - 1)
                        silu_store(r, c_own, 1, h)
                else:
                    if g <= 4:
                        c_rx = mod4(p - (g - 3)) if r == 0 else mod4(p + (g - 3))
                    else:
                        c_rx = mod4(p - 2) if r == 0 else mod4(p + 2)
                    silu_store(r, c_rx, (g + 1) % 2, h)
                d.wait_send()
                if k < NK - 2:
                    pl.semaphore_signal(
                        credit, inc=1, device_id=(peer_in,),
                        device_id_type=pl.DeviceIdType.MESH,
                    )
                if k + 2 < NK:
                    pl.semaphore_wait(credit, 1)
                    rdma(r, k + 2).start()

    return pl.pallas_call(
        body,
        out_shape=jax.ShapeDtypeStruct((M, N), BF16),
        in_specs=[
            pl.BlockSpec(memory_space=pl.ANY),
            pl.BlockSpec(memory_space=pl.ANY),
        ],
        out_specs=pl.BlockSpec(memory_space=pl.ANY),
        scratch_shapes=[
            pltpu.VMEM((2, CH, N), BF16),
            pltpu.VMEM((2, CH, N), BF16),
            pltpu.VMEM((K, N), BF16),
            pltpu.VMEM((2, WT, N), F32),
            pltpu.VMEM((HR, K), F32),
            pltpu.VMEM((HR, N), BF16),
            pltpu.SemaphoreType.DMA((2, 2)),
            pltpu.SemaphoreType.DMA((2, 2)),
            pltpu.SemaphoreType.DMA((2, 2)),
            pltpu.SemaphoreType.DMA((2, 2)),
            pltpu.SemaphoreType.REGULAR,
            pltpu.SemaphoreType.REGULAR,
            pltpu.SemaphoreType.DMA,
            pltpu.SemaphoreType.DMA((2,)),
            pltpu.SemaphoreType.DMA,
        ],
        compiler_params=pltpu.CompilerParams(
            collective_id=0,
            vmem_limit_bytes=100 * 1024 * 1024,
        ),
    )(x, w_mat)


# device time: 615655 ns/iter; 1.2705x vs baseline; 1.0035x over previous
import jax
import jax.numpy as jnp
from jax import lax
from jax.experimental import pallas as pl
from jax.experimental.pallas import tpu as pltpu

N_DEV = 4
M = 4096
K = 1024
N = 8192
HALF = M // 2
CH = HALF // N_DEV
HR = CH // 2
NT = 1024
WT = 128
NK = 12
F32 = jnp.float32
BF16 = jnp.bfloat16


def kernel(x, w_mat):
    def body(x_hbm, w_hbm, out_ref, comm_a, comm_b, wb, wstage, xstage, sbuf,
             send_a, recv_a, send_b, recv_b,
             credit_a, credit_b, out_sem, cvt_sems, x_sem):
        p = lax.axis_index("i")
        right = lax.rem(p + 1, N_DEV)
        left = lax.rem(p + N_DEV - 1, N_DEV)

        def mod4(v):
            return lax.rem(v + 2 * N_DEV, N_DEV)

        def ring(r):
            if r == 0:
                return comm_a, send_a, recv_a, credit_a, right, left
            return comm_b, send_b, recv_b, credit_b, left, right

        def base(r):
            return 0 if r == 0 else HALF

        def rdma(r, k):
            comm, ssem, rsem, _, peer, _ = ring(r)
            g, h = k // 2, k % 2
            rows = pl.ds(h * HR, HR)
            return pltpu.make_async_remote_copy(
                src_ref=comm.at[g % 2, rows, :],
                dst_ref=comm.at[(g + 1) % 2, rows, :],
                send_sem=ssem.at[g % 2, h],
                recv_sem=rsem.at[(g + 1) % 2, h],
                device_id=(peer,),
                device_id_type=pl.DeviceIdType.MESH,
            )

        xseq = []
        for k in (0, 1):
            for r in (0, 1):
                xseq.append(base(r) + p * CH + k * HR)
        for k in range(6):
            g, h = k // 2, k % 2
            for r in (0, 1):
                c_in = mod4(p - g - 1) if r == 0 else mod4(p + g + 1)
                xseq.append(base(r) + c_in * CH + h * HR)

        def x_copy(j):
            return pltpu.make_async_copy(
                x_hbm.at[pl.ds(xseq[j], HR), :],
                xstage.at[j % 2],
                x_sem.at[j % 2],
            )

        x_copy(0).start()
        x_copy(1).start()

        n_wt = K // WT
        cps = []
        for t in range(n_wt):
            cp = pltpu.make_async_copy(
                w_hbm.at[pl.ds(t * WT, WT), :],
                wstage.at[t % 2],
                cvt_sems.at[t % 2],
            )
            cp.start()
            cps.append(cp)
            if t >= 1:
                cps[t - 1].wait()
                wb[pl.ds((t - 1) * WT, WT), :] = wstage[(t - 1) % 2].astype(BF16)
        cps[n_wt - 1].wait()
        wb[pl.ds((n_wt - 1) * WT, WT), :] = wstage[(n_wt - 1) % 2].astype(BF16)

        def compute_sum(xi, r, slot, h, add):
            comm = ring(r)[0]
            x_copy(xi).wait()
            xc = xstage[xi % 2].astype(BF16)

            def tile(j, _):
                cols = pl.ds(pl.multiple_of(j * NT, NT), NT)
                part = jnp.dot(xc, wb[:, cols], preferred_element_type=F32)
                if add:
                    part = part + comm[slot, h * HR:(h + 1) * HR, cols].astype(F32)
                comm[slot, h * HR:(h + 1) * HR, cols] = part.astype(BF16)
                return 0

            lax.fori_loop(0, N // NT, tile, 0)
            if xi + 2 < len(xseq):
                x_copy(xi + 2).start()

        def silu_store(r, c, slot, h):
            comm = ring(r)[0]

            def tile(j, _):
                cols = pl.ds(pl.multiple_of(j * NT, NT), NT)
                y = comm[slot, h * HR:(h + 1) * HR, cols].astype(F32)
                sbuf[:, cols] = (y / (1.0 + jnp.exp(-y))).astype(BF16)
                return 0

            lax.fori_loop(0, N // NT, tile, 0)
            cp = pltpu.make_async_copy(
                sbuf,
                out_ref.at[pl.ds(base(r) + c * CH + h * HR, HR), :],
                out_sem,
            )
            cp.start()
            cp.wait()

        bsem = pltpu.get_barrier_semaphore()
        for nbr in (left, right):
            pl.semaphore_signal(
                bsem, inc=1, device_id=(nbr,),
                device_id_type=pl.DeviceIdType.MESH,
            )
        pl.semaphore_wait(bsem, 2)

        for k in (0, 1):
            for r in (0, 1):
                compute_sum(2 * k + r, r, 0, k, add=False)
                rdma(r, k).start()

        for k in range(NK):
            g, h = k // 2, k % 2
            for r in (0, 1):
                comm, _, _, credit, _, peer_in = ring(r)
                d = rdma(r, k)
                d.wait_recv()
                if g <= 2:
                    compute_sum(4 + 2 * k + r, r, (g + 1) % 2, h, add=True)
                    if g == 2:
                        c_own = mod4(p + 1) if r == 0 else mod4(p - 1)
                        silu_store(r, c_own, 1, h)
                else:
                    if g <= 4:
                        c_rx = mod4(p - (g - 3)) if r == 0 else mod4(p + (g - 3))
                    else:
                        c_rx = mod4(p - 2) if r == 0 else mod4(p + 2)
                    silu_store(r, c_rx, (g + 1) % 2, h)
                d.wait_send()
                if k < NK - 2:
                    pl.semaphore_signal(
                        credit, inc=1, device_id=(peer_in,),
                        device_id_type=pl.DeviceIdType.MESH,
                    )
                if k + 2 < NK:
                    pl.semaphore_wait(credit, 1)
                    rdma(r, k + 2).start()

    return pl.pallas_call(
        body,
        out_shape=jax.ShapeDtypeStruct((M, N), BF16),
        in_specs=[
            pl.BlockSpec(memory_space=pl.ANY),
            pl.BlockSpec(memory_space=pl.ANY),
        ],
        out_specs=pl.BlockSpec(memory_space=pl.ANY),
        scratch_shapes=[
            pltpu.VMEM((2, CH, N), BF16),
            pltpu.VMEM((2, CH, N), BF16),
            pltpu.VMEM((K, N), BF16),
            pltpu.VMEM((2, WT, N), F32),
            pltpu.VMEM((2, HR, K), F32),
            pltpu.VMEM((HR, N), BF16),
            pltpu.SemaphoreType.DMA((2, 2)),
            pltpu.SemaphoreType.DMA((2, 2)),
            pltpu.SemaphoreType.DMA((2, 2)),
            pltpu.SemaphoreType.DMA((2, 2)),
            pltpu.SemaphoreType.REGULAR,
            pltpu.SemaphoreType.REGULAR,
            pltpu.SemaphoreType.DMA,
            pltpu.SemaphoreType.DMA((2,)),
            pltpu.SemaphoreType.DMA((2,)),
        ],
        compiler_params=pltpu.CompilerParams(
            collective_id=0,
            vmem_limit_bytes=100 * 1024 * 1024,
        ),
    )(x, w_mat)
